# baseline (speedup 1.0000x reference)
import functools

import jax
import jax.numpy as jnp
from jax import lax
from jax.experimental import pallas as pl
from jax.experimental.pallas import tpu as pltpu
from jax.experimental.pallas import tpu_sc as plsc

NUM_CLASSES = 1000
HIDDEN = 1152
BATCH = 16384

N_TC = 6144
N_SC = BATCH - N_TC
TC_BLK = 256
K_PAD = 1024

_INFO = plsc.get_sparse_core_info()
NC = _INFO.num_cores
NS = _INFO.num_subcores
NW = NC * NS
B_PER_W = N_SC // NW
CHUNK = 16
NBUF = 3
NCHUNK = B_PER_W // CHUNK


def _sc_body(table_hbm, labels_hbm, out_hbm, idx_v, rows_a, rows_b, rows_c,
             gsem_a, gsem_b, gsem_c, wsem_a, wsem_b, wsem_c):
    wid = lax.axis_index("s") * NC + lax.axis_index("c")
    base = wid * B_PER_W

    pltpu.sync_copy(labels_hbm.at[pl.ds(base, B_PER_W)], idx_v)

    bufs = (rows_a, rows_b, rows_c)
    gsems = (gsem_a, gsem_b, gsem_c)
    wsems = (wsem_a, wsem_b, wsem_c)
    gcp = [None] * NBUF
    wcp = [None] * NBUF

    def gather(j):
        return pltpu.async_copy(
            table_hbm.at[idx_v.at[pl.ds(j * CHUNK, CHUNK)]],
            bufs[j % NBUF], gsems[j % NBUF])

    gcp[0] = gather(0)
    for i in range(NCHUNK):
        b = i % NBUF
        j = i + 1
        if j < NCHUNK:
            nb = j % NBUF
            if wcp[nb] is not None:
                wcp[nb].wait()
            gcp[nb] = gather(j)
        gcp[b].wait()
        wcp[b] = pltpu.async_copy(
            bufs[b], out_hbm.at[pl.ds(base + i * CHUNK, CHUNK)], wsems[b])
    for b in range(NBUF):
        if wcp[b] is not None:
            wcp[b].wait()


def _sc_gather(labels_sc, embedding_table):
    mesh = plsc.VectorSubcoreMesh(core_axis_name="c", subcore_axis_name="s")
    f = pl.kernel(
        _sc_body,
        out_type=jax.ShapeDtypeStruct((N_SC, HIDDEN), jnp.float32),
        mesh=mesh,
        scratch_types=[
            pltpu.VMEM((B_PER_W,), jnp.int32),
            pltpu.VMEM((CHUNK, HIDDEN), jnp.float32),
            pltpu.VMEM((CHUNK, HIDDEN), jnp.float32),
            pltpu.VMEM((CHUNK, HIDDEN), jnp.float32),
            pltpu.SemaphoreType.DMA,
            pltpu.SemaphoreType.DMA,
            pltpu.SemaphoreType.DMA,
            pltpu.SemaphoreType.DMA,
            pltpu.SemaphoreType.DMA,
            pltpu.SemaphoreType.DMA,
        ],
    )
    return f(embedding_table, labels_sc)


def _tc_body(labels_ref, table_ref, out_ref):
    labels = labels_ref[0]
    iota = lax.broadcasted_iota(jnp.int32, (TC_BLK, K_PAD), 1)
    one_hot = (labels == iota).astype(jnp.float32)
    out_ref[...] = jnp.dot(one_hot, table_ref[...],
                           preferred_element_type=jnp.float32)


def _tc_gather(labels_tc, table_pad):
    labels_3d = labels_tc.reshape(N_TC // TC_BLK, TC_BLK, 1)
    return pl.pallas_call(
        _tc_body,
        grid=(N_TC // TC_BLK,),
        in_specs=[
            pl.BlockSpec((1, TC_BLK, 1), lambda i: (i, 0, 0)),
            pl.BlockSpec((K_PAD, HIDDEN), lambda i: (0, 0)),
        ],
        out_specs=pl.BlockSpec((TC_BLK, HIDDEN), lambda i: (i, 0)),
        out_shape=jax.ShapeDtypeStruct((N_TC, HIDDEN), jnp.float32),
    )(labels_3d, table_pad)


@jax.jit
def _embed(labels, embedding_table):
    table_pad = jnp.pad(embedding_table,
                        ((0, K_PAD - NUM_CLASSES), (0, 0)))
    sc_out = _sc_gather(labels[:N_SC], embedding_table)
    tc_out = _tc_gather(labels[N_SC:], table_pad)
    return jnp.concatenate([sc_out, tc_out], axis=0)


def kernel(labels, embedding_table):
    return _embed(labels.astype(jnp.int32), embedding_table)


# speedup vs baseline: 1.7091x; 1.7091x over previous
import functools

import jax
import jax.numpy as jnp
from jax import lax
from jax.experimental import pallas as pl
from jax.experimental.pallas import tpu as pltpu
from jax.experimental.pallas import tpu_sc as plsc

NUM_CLASSES = 1000
HIDDEN = 1152
BATCH = 16384

_INFO = plsc.get_sparse_core_info()
NC = _INFO.num_cores
NS = _INFO.num_subcores
NW = NC * NS
B_PER_W = BATCH // NW
CHUNK = 32
NCHUNK = B_PER_W // CHUNK
NBUF = 3


def _embed_body(table_hbm, labels_hbm, out_hbm, idx_v, rows_a, rows_b, rows_c,
                gsem_a, gsem_b, gsem_c, wsem_a, wsem_b, wsem_c):
    wid = lax.axis_index("s") * NC + lax.axis_index("c")
    base = wid * B_PER_W

    pltpu.sync_copy(labels_hbm.at[pl.ds(base, B_PER_W)], idx_v)

    bufs = (rows_a, rows_b, rows_c)
    gsems = (gsem_a, gsem_b, gsem_c)
    wsems = (wsem_a, wsem_b, wsem_c)
    gcp = [None] * NBUF
    wcp = [None] * NBUF

    def gather(j):
        return pltpu.async_copy(
            table_hbm.at[idx_v.at[pl.ds(j * CHUNK, CHUNK)]],
            bufs[j % NBUF], gsems[j % NBUF])

    gcp[0] = gather(0)
    for i in range(NCHUNK):
        b = i % NBUF
        j = i + 1
        if j < NCHUNK:
            nb = j % NBUF
            if wcp[nb] is not None:
                wcp[nb].wait()
            gcp[nb] = gather(j)
        gcp[b].wait()
        wcp[b] = pltpu.async_copy(
            bufs[b], out_hbm.at[pl.ds(base + i * CHUNK, CHUNK)], wsems[b])
    for b in range(NBUF):
        if wcp[b] is not None:
            wcp[b].wait()


@jax.jit
def _embed(labels, embedding_table):
    mesh = plsc.VectorSubcoreMesh(core_axis_name="c", subcore_axis_name="s")
    f = pl.kernel(
        _embed_body,
        out_type=jax.ShapeDtypeStruct((BATCH, HIDDEN), jnp.float32),
        mesh=mesh,
        scratch_types=[
            pltpu.VMEM((B_PER_W,), jnp.int32),
            pltpu.VMEM((CHUNK, HIDDEN), jnp.float32),
            pltpu.VMEM((CHUNK, HIDDEN), jnp.float32),
            pltpu.VMEM((CHUNK, HIDDEN), jnp.float32),
            pltpu.SemaphoreType.DMA,
            pltpu.SemaphoreType.DMA,
            pltpu.SemaphoreType.DMA,
            pltpu.SemaphoreType.DMA,
            pltpu.SemaphoreType.DMA,
            pltpu.SemaphoreType.DMA,
        ],
    )
    return f(embedding_table, labels)


def kernel(labels, embedding_table):
    return _embed(labels.astype(jnp.int32), embedding_table)
